# baseline (speedup 1.0000x reference)
import functools

import jax
import jax.numpy as jnp
from jax import lax
from jax.experimental import pallas as pl
from jax.experimental.pallas import tpu as pltpu
from jax.experimental.pallas import tpu_sc as plsc

N = 10000
NPAD = 10112
D = 128
DH = 64
NG = 16
E = 320000
EPAD = 327680
NSUB = 16
EB = 128
NB = 160
RPT = NPAD // NSUB
NBLK = NPAD // 128
F32 = jnp.float32


def _leaky(v):
    return jnp.where(v >= 0.0, v, v * 0.2)


def _embed_body(x_ref, w_ref, asr_ref, adr_ref, h_ref, as_ref, ad_ref):
    h = jnp.dot(x_ref[...], w_ref[...], preferred_element_type=F32)
    h_ref[0] = h[:, 0:DH]
    h_ref[1] = h[:, DH:D]
    as_ref[...] = jnp.dot(h, asr_ref[...], preferred_element_type=F32)
    ad_ref[...] = jnp.dot(h, adr_ref[...], preferred_element_type=F32)


def _tc_embed(xp, W, a_s, a_d):
    return pl.pallas_call(
        _embed_body,
        grid=(NBLK,),
        in_specs=[
            pl.BlockSpec((128, D), lambda i: (i, 0)),
            pl.BlockSpec((D, D), lambda i: (0, 0)),
            pl.BlockSpec((D, 1), lambda i: (0, 0)),
            pl.BlockSpec((D, 1), lambda i: (0, 0)),
        ],
        out_specs=[
            pl.BlockSpec((2, 128, DH), lambda i: (0, i, 0)),
            pl.BlockSpec((128, 1), lambda i: (i, 0)),
            pl.BlockSpec((128, 1), lambda i: (i, 0)),
        ],
        out_shape=[
            jax.ShapeDtypeStruct((2, NPAD, DH), F32),
            jax.ShapeDtypeStruct((NPAD, 1), F32),
            jax.ShapeDtypeStruct((NPAD, 1), F32),
        ],
    )(xp, W, a_s, a_d)


def _tables_body(as_ref, ad_ref, c_ref, es_ref):
    a = as_ref[...]
    d = ad_ref[...]
    m = jnp.max(a[0:N, :])
    row = lax.broadcasted_iota(jnp.int32, (NPAD, 1), 0)
    c = jnp.where(row < N, _leaky(m + d), F32(1e30))
    c_ref[...] = c
    es_ref[...] = jnp.exp(_leaky(a + d) - c)


def _tc_tables(as1, ad1):
    return pl.pallas_call(
        _tables_body,
        out_shape=[
            jax.ShapeDtypeStruct((NPAD, 1), F32),
            jax.ShapeDtypeStruct((NPAD, 1), F32),
        ],
    )(as1, ad1)


def _combine(o_ref, d_ref, h_ref, es_ref, b_ref):
    num = jnp.concatenate([o_ref[0], o_ref[1]], axis=1)
    h1 = jnp.concatenate([h_ref[0], h_ref[1]], axis=1)
    es = es_ref[...]
    num = num + es * h1
    den = d_ref[...] + es + 1e-16
    v = num / den + b_ref[...]
    return jnp.where(v > 0.0, v, jnp.exp(v) - 1.0)


def _comb_body(o_ref, d_ref, h_ref, es_ref, b_ref, w_ref, asr_ref, adr_ref,
               h2_ref, as2_ref, ad2_ref):
    h1f = _combine(o_ref, d_ref, h_ref, es_ref, b_ref)
    h2 = jnp.dot(h1f, w_ref[...], preferred_element_type=F32)
    h2_ref[0] = h2[:, 0:DH]
    h2_ref[1] = h2[:, DH:D]
    as2_ref[...] = jnp.dot(h2, asr_ref[...], preferred_element_type=F32)
    ad2_ref[...] = jnp.dot(h2, adr_ref[...], preferred_element_type=F32)


def _tc_comb(o1, d1, h1p, es1, b, W, a_s, a_d):
    return pl.pallas_call(
        _comb_body,
        grid=(NBLK,),
        in_specs=[
            pl.BlockSpec((2, 128, DH), lambda i: (0, i, 0)),
            pl.BlockSpec((128, 1), lambda i: (i, 0)),
            pl.BlockSpec((2, 128, DH), lambda i: (0, i, 0)),
            pl.BlockSpec((128, 1), lambda i: (i, 0)),
            pl.BlockSpec((1, D), lambda i: (0, 0)),
            pl.BlockSpec((D, D), lambda i: (0, 0)),
            pl.BlockSpec((D, 1), lambda i: (0, 0)),
            pl.BlockSpec((D, 1), lambda i: (0, 0)),
        ],
        out_specs=[
            pl.BlockSpec((2, 128, DH), lambda i: (0, i, 0)),
            pl.BlockSpec((128, 1), lambda i: (i, 0)),
            pl.BlockSpec((128, 1), lambda i: (i, 0)),
        ],
        out_shape=[
            jax.ShapeDtypeStruct((2, NPAD, DH), F32),
            jax.ShapeDtypeStruct((NPAD, 1), F32),
            jax.ShapeDtypeStruct((NPAD, 1), F32),
        ],
    )(o1, d1, h1p, es1, b, W, a_s, a_d)


def _pool_body(o_ref, d_ref, h_ref, es_ref, b_ref, bt_ref, sum_ref, cnt_ref):
    h2f = _combine(o_ref, d_ref, h_ref, es_ref, b_ref)
    bt = bt_ref[0]
    g = lax.broadcasted_iota(jnp.int32, (NG, 128), 0)
    oh = (g == bt).astype(F32)
    psum = jnp.dot(oh, h2f, preferred_element_type=F32)
    pcnt = jnp.broadcast_to(jnp.sum(oh, axis=1, keepdims=True), (NG, D))
    i = pl.program_id(0)

    @pl.when(i == 0)
    def _():
        sum_ref[...] = psum
        cnt_ref[...] = pcnt

    @pl.when(i != 0)
    def _():
        sum_ref[...] += psum
        cnt_ref[...] += pcnt


def _tc_pool(o2, d2, h2p, es2, b, batchp):
    return pl.pallas_call(
        _pool_body,
        grid=(NBLK,),
        in_specs=[
            pl.BlockSpec((2, 128, DH), lambda i: (0, i, 0)),
            pl.BlockSpec((128, 1), lambda i: (i, 0)),
            pl.BlockSpec((2, 128, DH), lambda i: (0, i, 0)),
            pl.BlockSpec((128, 1), lambda i: (i, 0)),
            pl.BlockSpec((1, D), lambda i: (0, 0)),
            pl.BlockSpec((1, 1, 128), lambda i: (i, 0, 0)),
        ],
        out_specs=[
            pl.BlockSpec((NG, D), lambda i: (0, 0)),
            pl.BlockSpec((NG, D), lambda i: (0, 0)),
        ],
        out_shape=[
            jax.ShapeDtypeStruct((NG, D), F32),
            jax.ShapeDtypeStruct((NG, D), F32),
        ],
    )(o2, d2, h2p, es2, b, batchp)


def _head_body(s_ref, c_ref, wc_ref, bc_ref, o_ref):
    cnt = c_ref[...][:, 0:1]
    pooled = s_ref[...] / jnp.maximum(cnt, 1.0)
    lg = jnp.dot(pooled, wc_ref[...], preferred_element_type=F32) + bc_ref[...]
    o_ref[...] = 1.0 / (1.0 + jnp.exp(-lg))


def _tc_head(sums, cnts, Wc, bc):
    return pl.pallas_call(
        _head_body,
        out_shape=jax.ShapeDtypeStruct((NG, 1), F32),
    )(sums, cnts, Wc, bc)


def _sc_body(h_hbm, src_hbm, dst_hbm, as_hbm, ad_hbm, c_hbm,
             out_hbm, den_hbm,
             src_v, dst_v, as_v, ad_v, c_v, r_a, r_b, w_a, w_b, den_out,
             out_acc, den_acc, gs_a, gs_b, ss_a, ss_b):
    cid = lax.axis_index("c")
    sid = lax.axis_index("s")

    pltpu.sync_copy(src_hbm.at[sid], src_v)
    pltpu.sync_copy(dst_hbm.at[sid], dst_v)
    pltpu.sync_copy(as_hbm, as_v)
    pltpu.sync_copy(ad_hbm, ad_v)
    pltpu.sync_copy(c_hbm, c_v)

    zero = jnp.zeros((16,), F32)

    def _zero_row(r, carry):
        for cc in range(DH // 16):
            r_a[r, pl.ds(cc * 16, 16)] = zero
        return carry

    lax.fori_loop(0, EB, _zero_row, 0)
    for cc in range(EB // 16):
        w_a[pl.ds(cc * 16, 16)] = zero

    base = sid * RPT
    for k in range(4):
        pltpu.sync_copy(r_a, out_acc.at[pl.ds(base + k * 128, 128)])
        pltpu.sync_copy(w_a, den_acc.at[pl.ds(base + k * 128, 128)])
    pltpu.sync_copy(r_a.at[pl.ds(0, RPT - 512)],
                    out_acc.at[pl.ds(base + 512, RPT - 512)])
    pltpu.sync_copy(w_a.at[pl.ds(0, RPT - 512)],
                    den_acc.at[pl.ds(base + 512, RPT - 512)])
    plsc.subcore_barrier()

    def compute_w(j, wref):
        for k in range(EB // 16):
            sl = pl.ds(k * 16, 16)
            sv = src_v[j, sl]
            dv = dst_v[j, sl]
            a1 = plsc.load_gather(as_v, [sv])
            a2 = plsc.load_gather(ad_v, [dv])
            cg = plsc.load_gather(c_v, [dv])
            al = a1 + a2
            al = jnp.where(al >= 0.0, al, al * 0.2)
            wref[sl] = jnp.exp(al - cg)

    def scale(rows, wref):
        for e in range(EB):
            we = plsc.load_gather(wref, [jnp.full((16,), e, jnp.int32)])
            for cc in range(DH // 16):
                sl = pl.ds(cc * 16, 16)
                rows[e, sl] = rows[e, sl] * we

    def gstart(j, rows, sem):
        pltpu.async_copy(h_hbm.at[cid].at[src_v.at[j]], rows, sem)

    def gwait(j, rows, sem):
        pltpu.make_async_copy(h_hbm.at[cid].at[src_v.at[j]], rows, sem).wait()

    def scat_start(j, rows, wref, sem):
        pltpu.async_copy(rows, out_acc.at[dst_v.at[j]], sem, add=True)

        @pl.when(cid == 0)
        def _():
            pltpu.async_copy(wref, den_acc.at[dst_v.at[j]], sem, add=True)

    def scat_wait(j, rows, wref, sem):
        pltpu.make_async_copy(rows, out_acc.at[dst_v.at[j]], sem).wait()

        @pl.when(cid == 0)
        def _():
            pltpu.make_async_copy(wref, den_acc.at[dst_v.at[j]], sem).wait()

    gstart(0, r_a, gs_a)
    gstart(1, r_b, gs_b)

    def body(jj, carry):
        j0 = jj * 2
        j1 = j0 + 1
        compute_w(j0, w_a)
        gwait(j0, r_a, gs_a)
        scale(r_a, w_a)
        scat_start(j0, r_a, w_a, ss_a)

        compute_w(j1, w_b)
        gwait(j1, r_b, gs_b)
        scale(r_b, w_b)
        scat_start(j1, r_b, w_b, ss_b)

        scat_wait(j0, r_a, w_a, ss_a)

        @pl.when(jj < NB // 2 - 1)
        def _():
            gstart(j0 + 2, r_a, gs_a)

        scat_wait(j1, r_b, w_b, ss_b)

        @pl.when(jj < NB // 2 - 1)
        def _():
            gstart(j1 + 2, r_b, gs_b)

        return carry

    lax.fori_loop(0, NB // 2, body, 0)
    plsc.subcore_barrier()

    pltpu.sync_copy(out_acc.at[pl.ds(base, RPT)],
                    out_hbm.at[cid, pl.ds(base, RPT)])

    @pl.when(cid == 0)
    def _():
        pltpu.sync_copy(den_acc.at[pl.ds(base, RPT)], den_out)
        pltpu.sync_copy(den_out, den_hbm.at[pl.ds(base, RPT)])


@functools.partial(
    pl.kernel,
    out_type=[
        jax.ShapeDtypeStruct((2, NPAD, DH), F32),
        jax.ShapeDtypeStruct((NPAD,), F32),
    ],
    mesh=plsc.VectorSubcoreMesh(core_axis_name="c", subcore_axis_name="s"),
    compiler_params=pltpu.CompilerParams(
        needs_layout_passes=False, use_tc_tiling_on_sc=False),
    scratch_types=[
        pltpu.VMEM((NB, EB), jnp.int32),
        pltpu.VMEM((NB, EB), jnp.int32),
        pltpu.VMEM((NPAD,), F32),
        pltpu.VMEM((NPAD,), F32),
        pltpu.VMEM((NPAD,), F32),
        pltpu.VMEM((EB, DH), F32),
        pltpu.VMEM((EB, DH), F32),
        pltpu.VMEM((EB,), F32),
        pltpu.VMEM((EB,), F32),
        pltpu.VMEM((RPT,), F32),
        pltpu.VMEM_SHARED((NPAD, DH), F32),
        pltpu.VMEM_SHARED((NPAD,), F32),
        pltpu.SemaphoreType.DMA,
        pltpu.SemaphoreType.DMA,
        pltpu.SemaphoreType.DMA,
        pltpu.SemaphoreType.DMA,
    ],
)
def _sc_edge(h_hbm, src_hbm, dst_hbm, as_hbm, ad_hbm, c_hbm,
             out_hbm, den_hbm, *rest):
    _sc_body(h_hbm, src_hbm, dst_hbm, as_hbm, ad_hbm, c_hbm,
             out_hbm, den_hbm, *rest)


def kernel(x, edge_index, batch, W1, a_src1, a_dst1, b1,
           W2, a_src2, a_dst2, b2, Wc, bc):
    src = edge_index[0]
    dst = edge_index[1]
    pad = (N + (jnp.arange(EPAD - E, dtype=jnp.int32) % (NPAD - N))).astype(jnp.int32)
    srcp = jnp.concatenate([src, pad]).reshape(NSUB, NB, EB)
    dstp = jnp.concatenate([dst, pad]).reshape(NSUB, NB, EB)
    xp = jnp.concatenate([x, jnp.zeros((NPAD - N, D), F32)])
    batchp = jnp.concatenate(
        [batch, jnp.full((NPAD - N,), NG, jnp.int32)]).reshape(NBLK, 1, 128)

    h1p, as1, ad1 = _tc_embed(xp, W1, a_src1.reshape(D, 1), a_dst1.reshape(D, 1))
    c1, es1 = _tc_tables(as1, ad1)
    o1, d1 = _sc_edge(h1p, srcp, dstp,
                      as1.reshape(NPAD), ad1.reshape(NPAD), c1.reshape(NPAD))
    h2p, as2, ad2 = _tc_comb(o1, d1.reshape(NPAD, 1), h1p, es1,
                             b1.reshape(1, D), W2,
                             a_src2.reshape(D, 1), a_dst2.reshape(D, 1))
    c2, es2 = _tc_tables(as2, ad2)
    o2, d2 = _sc_edge(h2p, srcp, dstp,
                      as2.reshape(NPAD), ad2.reshape(NPAD), c2.reshape(NPAD))
    sums, cnts = _tc_pool(o2, d2.reshape(NPAD, 1), h2p, es2,
                          b2.reshape(1, D), batchp)
    sig = _tc_head(sums, cnts, Wc, bc.reshape(1, 1))
    return sig.reshape(NG)


# speedup vs baseline: 1.0347x; 1.0347x over previous
import functools

import jax
import jax.numpy as jnp
from jax import lax
from jax.experimental import pallas as pl
from jax.experimental.pallas import tpu as pltpu
from jax.experimental.pallas import tpu_sc as plsc

N = 10000
NPAD = 10112
D = 128
DH = 64
NG = 16
E = 320000
EPAD = 327680
NSUB = 16
EB = 128
NB = 160
RPT = NPAD // NSUB
NBLK = NPAD // 128
F32 = jnp.float32


def _leaky(v):
    return jnp.where(v >= 0.0, v, v * 0.2)


def _store_logits(i, h, asr_ref, adr_ref, as_ref, ad_ref):
    arow = i * 128 + lax.broadcasted_iota(jnp.int32, (128, 1), 0)
    av = jnp.dot(h, asr_ref[...], preferred_element_type=F32)
    as_ref[...] = jnp.where(arow < N, av, F32(-1e30))
    ad_ref[...] = jnp.dot(h, adr_ref[...], preferred_element_type=F32)


def _embed_body(x_ref, w_ref, asr_ref, adr_ref, h_ref, as_ref, ad_ref):
    h = jnp.dot(x_ref[...], w_ref[...], preferred_element_type=F32)
    h_ref[0] = h[:, 0:DH]
    h_ref[1] = h[:, DH:D]
    _store_logits(pl.program_id(0), h, asr_ref, adr_ref, as_ref, ad_ref)


def _tc_embed(xp, W, a_s, a_d):
    return pl.pallas_call(
        _embed_body,
        grid=(NBLK,),
        in_specs=[
            pl.BlockSpec((128, D), lambda i: (i, 0)),
            pl.BlockSpec((D, D), lambda i: (0, 0)),
            pl.BlockSpec((D, 1), lambda i: (0, 0)),
            pl.BlockSpec((D, 1), lambda i: (0, 0)),
        ],
        out_specs=[
            pl.BlockSpec((2, 128, DH), lambda i: (0, i, 0)),
            pl.BlockSpec((128, 1), lambda i: (i, 0)),
            pl.BlockSpec((128, 1), lambda i: (i, 0)),
        ],
        out_shape=[
            jax.ShapeDtypeStruct((2, NPAD, DH), F32),
            jax.ShapeDtypeStruct((NPAD, 1), F32),
            jax.ShapeDtypeStruct((NPAD, 1), F32),
        ],
    )(xp, W, a_s, a_d)


def _tables_body(as_ref, ad_ref, es_ref, m_ref):
    a = as_ref[...]
    d = ad_ref[...]
    m = jnp.max(a[0:N, :])
    c = _leaky(m + d)
    es_ref[...] = jnp.exp(_leaky(a + d) - c)
    m_ref[...] = jnp.full((1, 128), m, F32)


def _tc_tables(as1, ad1):
    return pl.pallas_call(
        _tables_body,
        out_shape=[
            jax.ShapeDtypeStruct((NPAD, 1), F32),
            jax.ShapeDtypeStruct((1, 128), F32),
        ],
    )(as1, ad1)


def _combine(o_ref, d_ref, h_ref, es_ref, b_ref):
    num = jnp.concatenate([o_ref[0], o_ref[1]], axis=1)
    h1 = jnp.concatenate([h_ref[0], h_ref[1]], axis=1)
    es = es_ref[...]
    num = num + es * h1
    den = d_ref[...] + es + 1e-16
    v = num / den + b_ref[...]
    return jnp.where(v > 0.0, v, jnp.exp(v) - 1.0)


def _comb_body(o_ref, d_ref, h_ref, es_ref, b_ref, w_ref, asr_ref, adr_ref,
               h2_ref, as2_ref, ad2_ref):
    h1f = _combine(o_ref, d_ref, h_ref, es_ref, b_ref)
    h2 = jnp.dot(h1f, w_ref[...], preferred_element_type=F32)
    h2_ref[0] = h2[:, 0:DH]
    h2_ref[1] = h2[:, DH:D]
    _store_logits(pl.program_id(0), h2, asr_ref, adr_ref, as2_ref, ad2_ref)


def _tc_comb(o1, d1, h1p, es1, b, W, a_s, a_d):
    return pl.pallas_call(
        _comb_body,
        grid=(NBLK,),
        in_specs=[
            pl.BlockSpec((2, 128, DH), lambda i: (0, i, 0)),
            pl.BlockSpec((128, 1), lambda i: (i, 0)),
            pl.BlockSpec((2, 128, DH), lambda i: (0, i, 0)),
            pl.BlockSpec((128, 1), lambda i: (i, 0)),
            pl.BlockSpec((1, D), lambda i: (0, 0)),
            pl.BlockSpec((D, D), lambda i: (0, 0)),
            pl.BlockSpec((D, 1), lambda i: (0, 0)),
            pl.BlockSpec((D, 1), lambda i: (0, 0)),
        ],
        out_specs=[
            pl.BlockSpec((2, 128, DH), lambda i: (0, i, 0)),
            pl.BlockSpec((128, 1), lambda i: (i, 0)),
            pl.BlockSpec((128, 1), lambda i: (i, 0)),
        ],
        out_shape=[
            jax.ShapeDtypeStruct((2, NPAD, DH), F32),
            jax.ShapeDtypeStruct((NPAD, 1), F32),
            jax.ShapeDtypeStruct((NPAD, 1), F32),
        ],
    )(o1, d1, h1p, es1, b, W, a_s, a_d)


def _pool_body(o_ref, d_ref, h_ref, es_ref, b_ref, bt_ref, sum_ref, cnt_ref):
    h2f = _combine(o_ref, d_ref, h_ref, es_ref, b_ref)
    bt = bt_ref[0]
    g = lax.broadcasted_iota(jnp.int32, (NG, 128), 0)
    oh = (g == bt).astype(F32)
    psum = jnp.dot(oh, h2f, preferred_element_type=F32)
    pcnt = jnp.broadcast_to(jnp.sum(oh, axis=1, keepdims=True), (NG, D))
    i = pl.program_id(0)

    @pl.when(i == 0)
    def _():
        sum_ref[...] = psum
        cnt_ref[...] = pcnt

    @pl.when(i != 0)
    def _():
        sum_ref[...] += psum
        cnt_ref[...] += pcnt


def _tc_pool(o2, d2, h2p, es2, b, batchp):
    return pl.pallas_call(
        _pool_body,
        grid=(NBLK,),
        in_specs=[
            pl.BlockSpec((2, 128, DH), lambda i: (0, i, 0)),
            pl.BlockSpec((128, 1), lambda i: (i, 0)),
            pl.BlockSpec((2, 128, DH), lambda i: (0, i, 0)),
            pl.BlockSpec((128, 1), lambda i: (i, 0)),
            pl.BlockSpec((1, D), lambda i: (0, 0)),
            pl.BlockSpec((1, 1, 128), lambda i: (i, 0, 0)),
        ],
        out_specs=[
            pl.BlockSpec((NG, D), lambda i: (0, 0)),
            pl.BlockSpec((NG, D), lambda i: (0, 0)),
        ],
        out_shape=[
            jax.ShapeDtypeStruct((NG, D), F32),
            jax.ShapeDtypeStruct((NG, D), F32),
        ],
    )(o2, d2, h2p, es2, b, batchp)


def _head_body(s_ref, c_ref, wc_ref, bc_ref, o_ref):
    cnt = c_ref[...][:, 0:1]
    pooled = s_ref[...] / jnp.maximum(cnt, 1.0)
    lg = jnp.dot(pooled, wc_ref[...], preferred_element_type=F32) + bc_ref[...]
    o_ref[...] = 1.0 / (1.0 + jnp.exp(-lg))


def _tc_head(sums, cnts, Wc, bc):
    return pl.pallas_call(
        _head_body,
        out_shape=jax.ShapeDtypeStruct((NG, 1), F32),
    )(sums, cnts, Wc, bc)


def _sc_body(h_hbm, src_hbm, dst_hbm, as_hbm, ad_hbm, m_hbm,
             out_hbm, den_hbm,
             src_v, dst_v, as_v, ad_v, m_v, r_a, r_b, s_buf,
             w_a, w_b, den_out,
             out_acc, den_acc, gs_a, gs_b):
    cid = lax.axis_index("c")
    sid = lax.axis_index("s")

    pltpu.sync_copy(src_hbm.at[sid], src_v)
    pltpu.sync_copy(dst_hbm.at[sid], dst_v)
    pltpu.sync_copy(as_hbm, as_v)
    pltpu.sync_copy(ad_hbm, ad_v)
    pltpu.sync_copy(m_hbm, m_v)

    zero = jnp.zeros((16,), F32)

    def _zero_row(r, carry):
        for cc in range(DH // 16):
            r_a[r, pl.ds(cc * 16, 16)] = zero
        return carry

    lax.fori_loop(0, EB, _zero_row, 0)
    for cc in range(EB // 16):
        w_a[pl.ds(cc * 16, 16)] = zero

    base = sid * RPT
    for k in range(4):
        pltpu.sync_copy(r_a, out_acc.at[pl.ds(base + k * 128, 128)])
        pltpu.sync_copy(w_a, den_acc.at[pl.ds(base + k * 128, 128)])
    pltpu.sync_copy(r_a.at[pl.ds(0, RPT - 512)],
                    out_acc.at[pl.ds(base + 512, RPT - 512)])
    pltpu.sync_copy(w_a.at[pl.ds(0, RPT - 512)],
                    den_acc.at[pl.ds(base + 512, RPT - 512)])
    plsc.subcore_barrier()

    def compute_w(j, wref):
        mv = m_v[pl.ds(0, 16)]
        for k in range(EB // 16):
            sl = pl.ds(k * 16, 16)
            sv = src_v[j, sl]
            dv = dst_v[j, sl]
            a1 = plsc.load_gather(as_v, [sv])
            a2 = plsc.load_gather(ad_v, [dv])
            al = _leaky(a1 + a2)
            cg = _leaky(mv + a2)
            wref[sl] = jnp.exp(al - cg)

    def scale(rows, sbuf, wref):
        for e in range(EB):
            we = plsc.load_gather(wref, [jnp.full((16,), e, jnp.int32)])
            for cc in range(DH // 16):
                sl = pl.ds(cc * 16, 16)
                sbuf[e, sl] = rows[e, sl] * we

    def gstart(j, rows, sem):
        pltpu.async_copy(h_hbm.at[cid].at[src_v.at[j]], rows, sem)

    def gwait(j, rows, sem):
        pltpu.make_async_copy(h_hbm.at[cid].at[src_v.at[j]], rows, sem).wait()

    def scat(j, rows, wref):
        pltpu.sync_copy(rows, out_acc.at[dst_v.at[j]], add=True)

        @pl.when(cid == 0)
        def _():
            pltpu.sync_copy(wref, den_acc.at[dst_v.at[j]], add=True)

    gstart(0, r_a, gs_a)
    gstart(1, r_b, gs_b)

    def body(jj, carry):
        j0 = jj * 2
        j1 = j0 + 1
        compute_w(j0, w_a)
        gwait(j0, r_a, gs_a)
        scale(r_a, s_buf, w_a)

        @pl.when(jj < NB // 2 - 1)
        def _():
            gstart(j0 + 2, r_a, gs_a)

        scat(j0, s_buf, w_a)

        compute_w(j1, w_b)
        gwait(j1, r_b, gs_b)
        scale(r_b, s_buf, w_b)

        @pl.when(jj < NB // 2 - 1)
        def _():
            gstart(j1 + 2, r_b, gs_b)

        scat(j1, s_buf, w_b)

        return carry

    lax.fori_loop(0, NB // 2, body, 0)
    plsc.subcore_barrier()

    pltpu.sync_copy(out_acc.at[pl.ds(base, RPT)],
                    out_hbm.at[cid, pl.ds(base, RPT)])

    @pl.when(cid == 0)
    def _():
        pltpu.sync_copy(den_acc.at[pl.ds(base, RPT)], den_out)
        pltpu.sync_copy(den_out, den_hbm.at[pl.ds(base, RPT)])


@functools.partial(
    pl.kernel,
    out_type=[
        jax.ShapeDtypeStruct((2, NPAD, DH), F32),
        jax.ShapeDtypeStruct((NPAD,), F32),
    ],
    mesh=plsc.VectorSubcoreMesh(core_axis_name="c", subcore_axis_name="s"),
    compiler_params=pltpu.CompilerParams(
        needs_layout_passes=False, use_tc_tiling_on_sc=False),
    scratch_types=[
        pltpu.VMEM((NB, EB), jnp.int32),
        pltpu.VMEM((NB, EB), jnp.int32),
        pltpu.VMEM((NPAD,), F32),
        pltpu.VMEM((NPAD,), F32),
        pltpu.VMEM((16,), F32),
        pltpu.VMEM((EB, DH), F32),
        pltpu.VMEM((EB, DH), F32),
        pltpu.VMEM((EB, DH), F32),
        pltpu.VMEM((EB,), F32),
        pltpu.VMEM((EB,), F32),
        pltpu.VMEM((RPT,), F32),
        pltpu.VMEM_SHARED((NPAD, DH), F32),
        pltpu.VMEM_SHARED((NPAD,), F32),
        pltpu.SemaphoreType.DMA,
        pltpu.SemaphoreType.DMA,
    ],
)
def _sc_edge(h_hbm, src_hbm, dst_hbm, as_hbm, ad_hbm, m_hbm,
             out_hbm, den_hbm, *rest):
    _sc_body(h_hbm, src_hbm, dst_hbm, as_hbm, ad_hbm, m_hbm,
             out_hbm, den_hbm, *rest)


def kernel(x, edge_index, batch, W1, a_src1, a_dst1, b1,
           W2, a_src2, a_dst2, b2, Wc, bc):
    src = edge_index[0]
    dst = edge_index[1]
    pad = (N + (jnp.arange(EPAD - E, dtype=jnp.int32) % (NPAD - N))).astype(jnp.int32)
    srcp = jnp.concatenate([src, pad]).reshape(NSUB, NB, EB)
    dstp = jnp.concatenate([dst, pad]).reshape(NSUB, NB, EB)
    xp = jnp.concatenate([x, jnp.zeros((NPAD - N, D), F32)])
    batchp = jnp.concatenate(
        [batch, jnp.full((NPAD - N,), NG, jnp.int32)]).reshape(NBLK, 1, 128)

    h1p, as1, ad1 = _tc_embed(xp, W1, a_src1.reshape(D, 1), a_dst1.reshape(D, 1))
    es1, m1 = _tc_tables(as1, ad1)
    o1, d1 = _sc_edge(h1p, srcp, dstp,
                      as1.reshape(NPAD), ad1.reshape(NPAD), m1.reshape(128)[0:16])
    h2p, as2, ad2 = _tc_comb(o1, d1.reshape(NPAD, 1), h1p, es1,
                             b1.reshape(1, D), W2,
                             a_src2.reshape(D, 1), a_dst2.reshape(D, 1))
    es2, m2 = _tc_tables(as2, ad2)
    o2, d2 = _sc_edge(h2p, srcp, dstp,
                      as2.reshape(NPAD), ad2.reshape(NPAD), m2.reshape(128)[0:16])
    sums, cnts = _tc_pool(o2, d2.reshape(NPAD, 1), h2p, es2,
                          b2.reshape(1, D), batchp)
    sig = _tc_head(sums, cnts, Wc, bc.reshape(1, 1))
    return sig.reshape(NG)


# speedup vs baseline: 1.8680x; 1.8054x over previous
import functools

import jax
import jax.numpy as jnp
from jax import lax
from jax.experimental import pallas as pl
from jax.experimental.pallas import tpu as pltpu
from jax.experimental.pallas import tpu_sc as plsc

N = 10000
NPAD = 10112
D = 128
DH = 64
NG = 16
E = 320000
EPAD = 327680
NSUB = 16
EB = 128
NB = 160
RPT = NPAD // NSUB
NBLK = NPAD // 128
F32 = jnp.float32


def _leaky(v):
    return jnp.where(v >= 0.0, v, v * 0.2)


def _store_logits(i, h, asr_ref, adr_ref, as_ref, ad_ref):
    arow = i * 128 + lax.broadcasted_iota(jnp.int32, (128, 1), 0)
    av = jnp.dot(h, asr_ref[...], preferred_element_type=F32)
    as_ref[...] = jnp.where(arow < N, av, F32(-1e30))
    ad_ref[...] = jnp.dot(h, adr_ref[...], preferred_element_type=F32)


def _embed_body(x_ref, w_ref, asr_ref, adr_ref, h_ref, as_ref, ad_ref):
    h = jnp.dot(x_ref[...], w_ref[...], preferred_element_type=F32)
    h_ref[0] = h[:, 0:DH]
    h_ref[1] = h[:, DH:D]
    _store_logits(pl.program_id(0), h, asr_ref, adr_ref, as_ref, ad_ref)


def _tc_embed(xp, W, a_s, a_d):
    return pl.pallas_call(
        _embed_body,
        grid=(NBLK,),
        in_specs=[
            pl.BlockSpec((128, D), lambda i: (i, 0)),
            pl.BlockSpec((D, D), lambda i: (0, 0)),
            pl.BlockSpec((D, 1), lambda i: (0, 0)),
            pl.BlockSpec((D, 1), lambda i: (0, 0)),
        ],
        out_specs=[
            pl.BlockSpec((2, 128, DH), lambda i: (0, i, 0)),
            pl.BlockSpec((128, 1), lambda i: (i, 0)),
            pl.BlockSpec((128, 1), lambda i: (i, 0)),
        ],
        out_shape=[
            jax.ShapeDtypeStruct((2, NPAD, DH), F32),
            jax.ShapeDtypeStruct((NPAD, 1), F32),
            jax.ShapeDtypeStruct((NPAD, 1), F32),
        ],
    )(xp, W, a_s, a_d)


def _tables_body(as_ref, ad_ref, es_ref, m_ref):
    a = as_ref[...]
    d = ad_ref[...]
    m = jnp.max(a[0:N, :])
    c = _leaky(m + d)
    es_ref[...] = jnp.exp(_leaky(a + d) - c)
    m_ref[...] = jnp.full((1, 128), m, F32)


def _tc_tables(as1, ad1):
    return pl.pallas_call(
        _tables_body,
        out_shape=[
            jax.ShapeDtypeStruct((NPAD, 1), F32),
            jax.ShapeDtypeStruct((1, 128), F32),
        ],
    )(as1, ad1)


def _combine(o_ref, d_ref, h_ref, es_ref, b_ref):
    num = jnp.concatenate([o_ref[0], o_ref[1]], axis=1)
    h1 = jnp.concatenate([h_ref[0], h_ref[1]], axis=1)
    es = es_ref[...]
    num = num + es * h1
    den = d_ref[...] + es + 1e-16
    v = num / den + b_ref[...]
    return jnp.where(v > 0.0, v, jnp.exp(v) - 1.0)


def _comb_body(o_ref, d_ref, h_ref, es_ref, b_ref, w_ref, asr_ref, adr_ref,
               h2_ref, as2_ref, ad2_ref):
    h1f = _combine(o_ref, d_ref, h_ref, es_ref, b_ref)
    h2 = jnp.dot(h1f, w_ref[...], preferred_element_type=F32)
    h2_ref[0] = h2[:, 0:DH]
    h2_ref[1] = h2[:, DH:D]
    _store_logits(pl.program_id(0), h2, asr_ref, adr_ref, as2_ref, ad2_ref)


def _tc_comb(o1, d1, h1p, es1, b, W, a_s, a_d):
    return pl.pallas_call(
        _comb_body,
        grid=(NBLK,),
        in_specs=[
            pl.BlockSpec((2, 128, DH), lambda i: (0, i, 0)),
            pl.BlockSpec((128, 1), lambda i: (i, 0)),
            pl.BlockSpec((2, 128, DH), lambda i: (0, i, 0)),
            pl.BlockSpec((128, 1), lambda i: (i, 0)),
            pl.BlockSpec((1, D), lambda i: (0, 0)),
            pl.BlockSpec((D, D), lambda i: (0, 0)),
            pl.BlockSpec((D, 1), lambda i: (0, 0)),
            pl.BlockSpec((D, 1), lambda i: (0, 0)),
        ],
        out_specs=[
            pl.BlockSpec((2, 128, DH), lambda i: (0, i, 0)),
            pl.BlockSpec((128, 1), lambda i: (i, 0)),
            pl.BlockSpec((128, 1), lambda i: (i, 0)),
        ],
        out_shape=[
            jax.ShapeDtypeStruct((2, NPAD, DH), F32),
            jax.ShapeDtypeStruct((NPAD, 1), F32),
            jax.ShapeDtypeStruct((NPAD, 1), F32),
        ],
    )(o1, d1, h1p, es1, b, W, a_s, a_d)


def _pool_body(o_ref, d_ref, h_ref, es_ref, b_ref, bt_ref, sum_ref, cnt_ref):
    h2f = _combine(o_ref, d_ref, h_ref, es_ref, b_ref)
    bt = bt_ref[0]
    g = lax.broadcasted_iota(jnp.int32, (NG, 128), 0)
    oh = (g == bt).astype(F32)
    psum = jnp.dot(oh, h2f, preferred_element_type=F32)
    pcnt = jnp.broadcast_to(jnp.sum(oh, axis=1, keepdims=True), (NG, D))
    i = pl.program_id(0)

    @pl.when(i == 0)
    def _():
        sum_ref[...] = psum
        cnt_ref[...] = pcnt

    @pl.when(i != 0)
    def _():
        sum_ref[...] += psum
        cnt_ref[...] += pcnt


def _tc_pool(o2, d2, h2p, es2, b, batchp):
    return pl.pallas_call(
        _pool_body,
        grid=(NBLK,),
        in_specs=[
            pl.BlockSpec((2, 128, DH), lambda i: (0, i, 0)),
            pl.BlockSpec((128, 1), lambda i: (i, 0)),
            pl.BlockSpec((2, 128, DH), lambda i: (0, i, 0)),
            pl.BlockSpec((128, 1), lambda i: (i, 0)),
            pl.BlockSpec((1, D), lambda i: (0, 0)),
            pl.BlockSpec((1, 1, 128), lambda i: (i, 0, 0)),
        ],
        out_specs=[
            pl.BlockSpec((NG, D), lambda i: (0, 0)),
            pl.BlockSpec((NG, D), lambda i: (0, 0)),
        ],
        out_shape=[
            jax.ShapeDtypeStruct((NG, D), F32),
            jax.ShapeDtypeStruct((NG, D), F32),
        ],
    )(o2, d2, h2p, es2, b, batchp)


def _head_body(s_ref, c_ref, wc_ref, bc_ref, o_ref):
    cnt = c_ref[...][:, 0:1]
    pooled = s_ref[...] / jnp.maximum(cnt, 1.0)
    lg = jnp.dot(pooled, wc_ref[...], preferred_element_type=F32) + bc_ref[...]
    o_ref[...] = 1.0 / (1.0 + jnp.exp(-lg))


def _tc_head(sums, cnts, Wc, bc):
    return pl.pallas_call(
        _head_body,
        out_shape=jax.ShapeDtypeStruct((NG, 1), F32),
    )(sums, cnts, Wc, bc)


def _sc_body(h_hbm, src_hbm, dst_hbm, as_hbm, ad_hbm, m_hbm,
             out_hbm, den_hbm,
             src_v, dst_v, as_v, ad_v, m_v, r_a, r_b, s_buf,
             w_a, w_b, den_out,
             out_acc, den_acc, gs_a, gs_b):
    cid = lax.axis_index("c")
    sid = lax.axis_index("s")

    pltpu.sync_copy(src_hbm.at[sid], src_v)
    pltpu.sync_copy(dst_hbm.at[sid], dst_v)
    pltpu.sync_copy(as_hbm, as_v)
    pltpu.sync_copy(ad_hbm, ad_v)
    pltpu.sync_copy(m_hbm, m_v)

    zero = jnp.zeros((16,), F32)

    def _zero_row(r, carry):
        for cc in range(DH // 16):
            r_a[r, pl.ds(cc * 16, 16)] = zero
        return carry

    lax.fori_loop(0, EB, _zero_row, 0)
    for cc in range(EB // 16):
        w_a[pl.ds(cc * 16, 16)] = zero

    base = sid * RPT
    for k in range(4):
        pltpu.sync_copy(r_a, out_acc.at[pl.ds(base + k * 128, 128)])
        pltpu.sync_copy(w_a, den_acc.at[pl.ds(base + k * 128, 128)])
    pltpu.sync_copy(r_a.at[pl.ds(0, RPT - 512)],
                    out_acc.at[pl.ds(base + 512, RPT - 512)])
    pltpu.sync_copy(w_a.at[pl.ds(0, RPT - 512)],
                    den_acc.at[pl.ds(base + 512, RPT - 512)])
    plsc.subcore_barrier()

    def compute_w(j, wref):
        mv = m_v[pl.ds(0, 16)]
        for k in range(EB // 16):
            sl = pl.ds(k * 16, 16)
            sv = src_v[j, sl]
            dv = dst_v[j, sl]
            a1 = plsc.load_gather(as_v, [sv])
            a2 = plsc.load_gather(ad_v, [dv])
            al = _leaky(a1 + a2)
            cg = _leaky(mv + a2)
            wref[sl] = jnp.exp(al - cg)

    def scale(rows, sbuf, wref):
        for g in range(EB // 16):
            wv = wref[pl.ds(g * 16, 16)]
            for l in range(16):
                e = g * 16 + l
                we = jnp.take_along_axis(
                    wv, jnp.full((16,), l, jnp.int32), axis=0)
                for cc in range(DH // 16):
                    sl = pl.ds(cc * 16, 16)
                    sbuf[e, sl] = rows[e, sl] * we

    def gstart(j, rows, sem):
        pltpu.async_copy(h_hbm.at[cid].at[src_v.at[j]], rows, sem)

    def gwait(j, rows, sem):
        pltpu.make_async_copy(h_hbm.at[cid].at[src_v.at[j]], rows, sem).wait()

    def scat(j, rows, wref):
        pltpu.sync_copy(rows, out_acc.at[dst_v.at[j]], add=True)

        @pl.when(cid == 0)
        def _():
            pltpu.sync_copy(wref, den_acc.at[dst_v.at[j]], add=True)

    gstart(0, r_a, gs_a)
    gstart(1, r_b, gs_b)

    def body(jj, carry):
        j0 = jj * 2
        j1 = j0 + 1
        compute_w(j0, w_a)
        gwait(j0, r_a, gs_a)
        scale(r_a, s_buf, w_a)

        @pl.when(jj < NB // 2 - 1)
        def _():
            gstart(j0 + 2, r_a, gs_a)

        scat(j0, s_buf, w_a)

        compute_w(j1, w_b)
        gwait(j1, r_b, gs_b)
        scale(r_b, s_buf, w_b)

        @pl.when(jj < NB // 2 - 1)
        def _():
            gstart(j1 + 2, r_b, gs_b)

        scat(j1, s_buf, w_b)

        return carry

    lax.fori_loop(0, NB // 2, body, 0)
    plsc.subcore_barrier()

    pltpu.sync_copy(out_acc.at[pl.ds(base, RPT)],
                    out_hbm.at[cid, pl.ds(base, RPT)])

    @pl.when(cid == 0)
    def _():
        pltpu.sync_copy(den_acc.at[pl.ds(base, RPT)], den_out)
        pltpu.sync_copy(den_out, den_hbm.at[pl.ds(base, RPT)])


@functools.partial(
    pl.kernel,
    out_type=[
        jax.ShapeDtypeStruct((2, NPAD, DH), F32),
        jax.ShapeDtypeStruct((NPAD,), F32),
    ],
    mesh=plsc.VectorSubcoreMesh(core_axis_name="c", subcore_axis_name="s"),
    compiler_params=pltpu.CompilerParams(
        needs_layout_passes=False, use_tc_tiling_on_sc=False),
    scratch_types=[
        pltpu.VMEM((NB, EB), jnp.int32),
        pltpu.VMEM((NB, EB), jnp.int32),
        pltpu.VMEM((NPAD,), F32),
        pltpu.VMEM((NPAD,), F32),
        pltpu.VMEM((16,), F32),
        pltpu.VMEM((EB, DH), F32),
        pltpu.VMEM((EB, DH), F32),
        pltpu.VMEM((EB, DH), F32),
        pltpu.VMEM((EB,), F32),
        pltpu.VMEM((EB,), F32),
        pltpu.VMEM((RPT,), F32),
        pltpu.VMEM_SHARED((NPAD, DH), F32),
        pltpu.VMEM_SHARED((NPAD,), F32),
        pltpu.SemaphoreType.DMA,
        pltpu.SemaphoreType.DMA,
    ],
)
def _sc_edge(h_hbm, src_hbm, dst_hbm, as_hbm, ad_hbm, m_hbm,
             out_hbm, den_hbm, *rest):
    _sc_body(h_hbm, src_hbm, dst_hbm, as_hbm, ad_hbm, m_hbm,
             out_hbm, den_hbm, *rest)


def kernel(x, edge_index, batch, W1, a_src1, a_dst1, b1,
           W2, a_src2, a_dst2, b2, Wc, bc):
    src = edge_index[0]
    dst = edge_index[1]
    pad = (N + (jnp.arange(EPAD - E, dtype=jnp.int32) % (NPAD - N))).astype(jnp.int32)
    srcp = jnp.concatenate([src, pad]).reshape(NSUB, NB, EB)
    dstp = jnp.concatenate([dst, pad]).reshape(NSUB, NB, EB)
    xp = jnp.concatenate([x, jnp.zeros((NPAD - N, D), F32)])
    batchp = jnp.concatenate(
        [batch, jnp.full((NPAD - N,), NG, jnp.int32)]).reshape(NBLK, 1, 128)

    h1p, as1, ad1 = _tc_embed(xp, W1, a_src1.reshape(D, 1), a_dst1.reshape(D, 1))
    es1, m1 = _tc_tables(as1, ad1)
    o1, d1 = _sc_edge(h1p, srcp, dstp,
                      as1.reshape(NPAD), ad1.reshape(NPAD), m1.reshape(128)[0:16])
    h2p, as2, ad2 = _tc_comb(o1, d1.reshape(NPAD, 1), h1p, es1,
                             b1.reshape(1, D), W2,
                             a_src2.reshape(D, 1), a_dst2.reshape(D, 1))
    es2, m2 = _tc_tables(as2, ad2)
    o2, d2 = _sc_edge(h2p, srcp, dstp,
                      as2.reshape(NPAD), ad2.reshape(NPAD), m2.reshape(128)[0:16])
    sums, cnts = _tc_pool(o2, d2.reshape(NPAD, 1), h2p, es2,
                          b2.reshape(1, D), batchp)
    sig = _tc_head(sums, cnts, Wc, bc.reshape(1, 1))
    return sig.reshape(NG)


# speedup vs baseline: 2.3719x; 1.2698x over previous
import functools

import jax
import jax.numpy as jnp
from jax import lax
from jax.experimental import pallas as pl
from jax.experimental.pallas import tpu as pltpu
from jax.experimental.pallas import tpu_sc as plsc

N = 10000
NPAD = 10240
D = 128
DH = 64
NG = 16
E = 320000
EPAD = 327680
NSUB = 16
EB = 128
NB = 160
RPT = NPAD // NSUB
BR = 1280
NBLK = NPAD // BR
F32 = jnp.float32


def _leaky(v):
    return jnp.where(v >= 0.0, v, v * 0.2)


def _store_logits(i, h, asc_ref, adc_ref, asr_ref, adr_ref,
                  as_col, ad_col, as_row, ad_row):
    arow = i * BR + lax.broadcasted_iota(jnp.int32, (BR, 1), 0)
    av = jnp.dot(h, asc_ref[...], preferred_element_type=F32)
    as_col[...] = jnp.where(arow < N, av, F32(-1e30))
    ad_col[...] = jnp.dot(h, adc_ref[...], preferred_element_type=F32)
    dn = (((1,), (1,)), ((), ()))
    avr = lax.dot_general(asr_ref[...], h, dn, preferred_element_type=F32)
    lrow = i * BR + lax.broadcasted_iota(jnp.int32, (1, BR), 1)
    sl = pl.ds(i * BR, BR)
    as_row[:, sl] = jnp.where(lrow < N, avr, F32(-1e30))
    ad_row[:, sl] = lax.dot_general(adr_ref[...], h, dn,
                                    preferred_element_type=F32)


def _embed_body(x_ref, w_ref, asc_ref, adc_ref, asr_ref, adr_ref,
                h_ref, as_col, ad_col, as_row, ad_row):
    h = jnp.dot(x_ref[...], w_ref[...], preferred_element_type=F32)
    h_ref[0] = h[:, 0:DH]
    h_ref[1] = h[:, DH:D]
    _store_logits(pl.program_id(0), h, asc_ref, adc_ref, asr_ref, adr_ref,
                  as_col, ad_col, as_row, ad_row)


_LOGIT_OUT_SPECS = [
    pl.BlockSpec((BR, 1), lambda i: (i, 0)),
    pl.BlockSpec((BR, 1), lambda i: (i, 0)),
    pl.BlockSpec((1, NPAD), lambda i: (0, 0)),
    pl.BlockSpec((1, NPAD), lambda i: (0, 0)),
]
_LOGIT_OUT_SHAPE = [
    jax.ShapeDtypeStruct((NPAD, 1), F32),
    jax.ShapeDtypeStruct((NPAD, 1), F32),
    jax.ShapeDtypeStruct((1, NPAD), F32),
    jax.ShapeDtypeStruct((1, NPAD), F32),
]
_VEC_IN_SPECS = [
    pl.BlockSpec((D, 1), lambda i: (0, 0)),
    pl.BlockSpec((D, 1), lambda i: (0, 0)),
    pl.BlockSpec((1, D), lambda i: (0, 0)),
    pl.BlockSpec((1, D), lambda i: (0, 0)),
]


def _tc_embed(xp, W, a_sc, a_dc, a_sr, a_dr):
    return pl.pallas_call(
        _embed_body,
        grid=(NBLK,),
        in_specs=[
            pl.BlockSpec((BR, D), lambda i: (i, 0)),
            pl.BlockSpec((D, D), lambda i: (0, 0)),
            *_VEC_IN_SPECS,
        ],
        out_specs=[
            pl.BlockSpec((2, BR, DH), lambda i: (0, i, 0)),
            *_LOGIT_OUT_SPECS,
        ],
        out_shape=[
            jax.ShapeDtypeStruct((2, NPAD, DH), F32),
            *_LOGIT_OUT_SHAPE,
        ],
    )(xp, W, a_sc, a_dc, a_sr, a_dr)


def _tables_body(as_ref, ad_ref, es_ref, m_ref):
    a = as_ref[...]
    d = ad_ref[...]
    m = jnp.max(a[0:N, :])
    c = _leaky(m + d)
    es_ref[...] = jnp.exp(_leaky(a + d) - c)
    m_ref[...] = jnp.full((1, 128), m, F32)


def _tc_tables(as1, ad1):
    return pl.pallas_call(
        _tables_body,
        out_shape=[
            jax.ShapeDtypeStruct((NPAD, 1), F32),
            jax.ShapeDtypeStruct((1, 128), F32),
        ],
    )(as1, ad1)


def _combine(o_ref, d_ref, h_ref, es_ref, b_ref):
    num = jnp.concatenate([o_ref[0], o_ref[1]], axis=1)
    h1 = jnp.concatenate([h_ref[0], h_ref[1]], axis=1)
    es = es_ref[...]
    num = num + es * h1
    den = d_ref[...] + es + 1e-16
    v = num / den + b_ref[...]
    return jnp.where(v > 0.0, v, jnp.exp(v) - 1.0)


def _comb_body(o_ref, d_ref, h_ref, es_ref, b_ref, w_ref,
               asc_ref, adc_ref, asr_ref, adr_ref,
               h2_ref, as_col, ad_col, as_row, ad_row):
    h1f = _combine(o_ref, d_ref, h_ref, es_ref, b_ref)
    h2 = jnp.dot(h1f, w_ref[...], preferred_element_type=F32)
    h2_ref[0] = h2[:, 0:DH]
    h2_ref[1] = h2[:, DH:D]
    _store_logits(pl.program_id(0), h2, asc_ref, adc_ref, asr_ref, adr_ref,
                  as_col, ad_col, as_row, ad_row)


def _tc_comb(o1, d1, h1p, es1, b, W, a_sc, a_dc, a_sr, a_dr):
    return pl.pallas_call(
        _comb_body,
        grid=(NBLK,),
        in_specs=[
            pl.BlockSpec((2, BR, DH), lambda i: (0, i, 0)),
            pl.BlockSpec((BR, 1), lambda i: (i, 0)),
            pl.BlockSpec((2, BR, DH), lambda i: (0, i, 0)),
            pl.BlockSpec((BR, 1), lambda i: (i, 0)),
            pl.BlockSpec((1, D), lambda i: (0, 0)),
            pl.BlockSpec((D, D), lambda i: (0, 0)),
            *_VEC_IN_SPECS,
        ],
        out_specs=[
            pl.BlockSpec((2, BR, DH), lambda i: (0, i, 0)),
            *_LOGIT_OUT_SPECS,
        ],
        out_shape=[
            jax.ShapeDtypeStruct((2, NPAD, DH), F32),
            *_LOGIT_OUT_SHAPE,
        ],
    )(o1, d1, h1p, es1, b, W, a_sc, a_dc, a_sr, a_dr)


def _pool_body(o_ref, d_ref, h_ref, es_ref, b_ref, bt_ref, sum_ref, cnt_ref):
    h2f = _combine(o_ref, d_ref, h_ref, es_ref, b_ref)
    bt = bt_ref[0]
    g = lax.broadcasted_iota(jnp.int32, (NG, BR), 0)
    oh = (g == bt).astype(F32)
    psum = jnp.dot(oh, h2f, preferred_element_type=F32)
    pcnt = jnp.broadcast_to(jnp.sum(oh, axis=1, keepdims=True), (NG, D))
    i = pl.program_id(0)

    @pl.when(i == 0)
    def _():
        sum_ref[...] = psum
        cnt_ref[...] = pcnt

    @pl.when(i != 0)
    def _():
        sum_ref[...] += psum
        cnt_ref[...] += pcnt


def _tc_pool(o2, d2, h2p, es2, b, batchp):
    return pl.pallas_call(
        _pool_body,
        grid=(NBLK,),
        in_specs=[
            pl.BlockSpec((2, BR, DH), lambda i: (0, i, 0)),
            pl.BlockSpec((BR, 1), lambda i: (i, 0)),
            pl.BlockSpec((2, BR, DH), lambda i: (0, i, 0)),
            pl.BlockSpec((BR, 1), lambda i: (i, 0)),
            pl.BlockSpec((1, D), lambda i: (0, 0)),
            pl.BlockSpec((1, 1, BR), lambda i: (i, 0, 0)),
        ],
        out_specs=[
            pl.BlockSpec((NG, D), lambda i: (0, 0)),
            pl.BlockSpec((NG, D), lambda i: (0, 0)),
        ],
        out_shape=[
            jax.ShapeDtypeStruct((NG, D), F32),
            jax.ShapeDtypeStruct((NG, D), F32),
        ],
    )(o2, d2, h2p, es2, b, batchp)


def _head_body(s_ref, c_ref, wc_ref, bc_ref, o_ref):
    cnt = c_ref[...][:, 0:1]
    pooled = s_ref[...] / jnp.maximum(cnt, 1.0)
    lg = jnp.dot(pooled, wc_ref[...], preferred_element_type=F32) + bc_ref[...]
    o_ref[...] = 1.0 / (1.0 + jnp.exp(-lg))


def _tc_head(sums, cnts, Wc, bc):
    return pl.pallas_call(
        _head_body,
        out_shape=jax.ShapeDtypeStruct((NG, 1), F32),
    )(sums, cnts, Wc, bc)


def _sc_body(h_hbm, src_hbm, dst_hbm, as_hbm, ad_hbm, m_hbm,
             out_hbm, den_hbm,
             src_v, dst_v, as_v, ad_v, m_v, r_a, r_b, s_buf,
             w_a, w_b, den_out,
             out_acc, den_acc, gs_a, gs_b):
    cid = lax.axis_index("c")
    sid = lax.axis_index("s")

    pltpu.sync_copy(src_hbm.at[sid], src_v)
    pltpu.sync_copy(dst_hbm.at[sid], dst_v)
    pltpu.sync_copy(as_hbm.at[0], as_v)
    pltpu.sync_copy(ad_hbm.at[0], ad_v)
    pltpu.sync_copy(m_hbm.at[0, pl.ds(0, 16)], m_v)

    zero = jnp.zeros((16,), F32)

    def _zero_row(r, carry):
        for cc in range(DH // 16):
            r_a[r, pl.ds(cc * 16, 16)] = zero
        return carry

    lax.fori_loop(0, EB, _zero_row, 0)
    for cc in range(EB // 16):
        w_a[pl.ds(cc * 16, 16)] = zero

    base = sid * RPT
    for k in range(4):
        pltpu.sync_copy(r_a, out_acc.at[pl.ds(base + k * 128, 128)])
        pltpu.sync_copy(w_a, den_acc.at[pl.ds(base + k * 128, 128)])
    pltpu.sync_copy(r_a.at[pl.ds(0, RPT - 512)],
                    out_acc.at[pl.ds(base + 512, RPT - 512)])
    pltpu.sync_copy(w_a.at[pl.ds(0, RPT - 512)],
                    den_acc.at[pl.ds(base + 512, RPT - 512)])
    plsc.subcore_barrier()

    def compute_w(j, wref):
        mv = m_v[pl.ds(0, 16)]

        def wgroup(k, carry):
            sl = pl.ds(k * 16, 16)
            sv = src_v[j, sl]
            dv = dst_v[j, sl]
            a1 = plsc.load_gather(as_v, [sv])
            a2 = plsc.load_gather(ad_v, [dv])
            al = _leaky(a1 + a2)
            cg = _leaky(mv + a2)
            wref[sl] = jnp.exp(al - cg)
            return carry

        lax.fori_loop(0, EB // 16, wgroup, 0)

    def scale(rows, sbuf, wref):
        def sgroup(g, carry):
            wv = wref[pl.ds(g * 16, 16)]
            e0 = g * 16
            for l in range(16):
                we = jnp.take_along_axis(
                    wv, jnp.full((16,), l, jnp.int32), axis=0)
                for cc in range(DH // 16):
                    sl = pl.ds(cc * 16, 16)
                    sbuf[e0 + l, sl] = rows[e0 + l, sl] * we
            return carry

        lax.fori_loop(0, EB // 16, sgroup, 0)

    def gstart(j, rows, sem):
        pltpu.async_copy(h_hbm.at[cid].at[src_v.at[j]], rows, sem)

    def gwait(j, rows, sem):
        pltpu.make_async_copy(h_hbm.at[cid].at[src_v.at[j]], rows, sem).wait()

    def scat(j, rows, wref):
        pltpu.sync_copy(rows, out_acc.at[dst_v.at[j]], add=True)

        @pl.when(cid == 0)
        def _():
            pltpu.sync_copy(wref, den_acc.at[dst_v.at[j]], add=True)

    gstart(0, r_a, gs_a)
    gstart(1, r_b, gs_b)

    def body(jj, carry):
        j0 = jj * 2
        j1 = j0 + 1
        compute_w(j0, w_a)
        gwait(j0, r_a, gs_a)
        scale(r_a, s_buf, w_a)

        @pl.when(jj < NB // 2 - 1)
        def _():
            gstart(j0 + 2, r_a, gs_a)

        scat(j0, s_buf, w_a)

        compute_w(j1, w_b)
        gwait(j1, r_b, gs_b)
        scale(r_b, s_buf, w_b)

        @pl.when(jj < NB // 2 - 1)
        def _():
            gstart(j1 + 2, r_b, gs_b)

        scat(j1, s_buf, w_b)

        return carry

    lax.fori_loop(0, NB // 2, body, 0)
    plsc.subcore_barrier()

    pltpu.sync_copy(out_acc.at[pl.ds(base, RPT)],
                    out_hbm.at[cid, pl.ds(base, RPT)])

    @pl.when(cid == 0)
    def _():
        pltpu.sync_copy(den_acc.at[pl.ds(base, RPT)], den_out)
        pltpu.sync_copy(den_out, den_hbm.at[pl.ds(base, RPT)])


@functools.partial(
    pl.kernel,
    out_type=[
        jax.ShapeDtypeStruct((2, NPAD, DH), F32),
        jax.ShapeDtypeStruct((NPAD,), F32),
    ],
    mesh=plsc.VectorSubcoreMesh(core_axis_name="c", subcore_axis_name="s"),
    compiler_params=pltpu.CompilerParams(
        needs_layout_passes=False, use_tc_tiling_on_sc=False),
    scratch_types=[
        pltpu.VMEM((NB, EB), jnp.int32),
        pltpu.VMEM((NB, EB), jnp.int32),
        pltpu.VMEM((NPAD,), F32),
        pltpu.VMEM((NPAD,), F32),
        pltpu.VMEM((16,), F32),
        pltpu.VMEM((EB, DH), F32),
        pltpu.VMEM((EB, DH), F32),
        pltpu.VMEM((EB, DH), F32),
        pltpu.VMEM((EB,), F32),
        pltpu.VMEM((EB,), F32),
        pltpu.VMEM((RPT,), F32),
        pltpu.VMEM_SHARED((NPAD, DH), F32),
        pltpu.VMEM_SHARED((NPAD,), F32),
        pltpu.SemaphoreType.DMA,
        pltpu.SemaphoreType.DMA,
    ],
)
def _sc_edge(h_hbm, src_hbm, dst_hbm, as_hbm, ad_hbm, m_hbm,
             out_hbm, den_hbm, *rest):
    _sc_body(h_hbm, src_hbm, dst_hbm, as_hbm, ad_hbm, m_hbm,
             out_hbm, den_hbm, *rest)


def kernel(x, edge_index, batch, W1, a_src1, a_dst1, b1,
           W2, a_src2, a_dst2, b2, Wc, bc):
    src = edge_index[0]
    dst = edge_index[1]
    pad = (N + (jnp.arange(EPAD - E, dtype=jnp.int32) % (NPAD - N))).astype(jnp.int32)
    srcp = jnp.concatenate([src, pad]).reshape(NSUB, NB, EB)
    dstp = jnp.concatenate([dst, pad]).reshape(NSUB, NB, EB)
    xp = jnp.concatenate([x, jnp.zeros((NPAD - N, D), F32)])
    batchp = jnp.concatenate(
        [batch, jnp.full((NPAD - N,), NG, jnp.int32)]).reshape(NBLK, 1, BR)

    h1p, as1c, ad1c, as1r, ad1r = _tc_embed(
        xp, W1, a_src1.reshape(D, 1), a_dst1.reshape(D, 1), a_src1, a_dst1)
    es1, m1 = _tc_tables(as1c, ad1c)
    o1, d1 = _sc_edge(h1p, srcp, dstp, as1r, ad1r, m1)
    h2p, as2c, ad2c, as2r, ad2r = _tc_comb(
        o1, d1.reshape(NPAD, 1), h1p, es1, b1.reshape(1, D), W2,
        a_src2.reshape(D, 1), a_dst2.reshape(D, 1), a_src2, a_dst2)
    es2, m2 = _tc_tables(as2c, ad2c)
    o2, d2 = _sc_edge(h2p, srcp, dstp, as2r, ad2r, m2)
    sums, cnts = _tc_pool(o2, d2.reshape(NPAD, 1), h2p, es2,
                          b2.reshape(1, D), batchp)
    sig = _tc_head(sums, cnts, Wc, bc.reshape(1, 1))
    return sig.reshape(NG)


# speedup vs baseline: 2.7630x; 1.1649x over previous
import functools

import jax
import jax.numpy as jnp
from jax import lax
from jax.experimental import pallas as pl
from jax.experimental.pallas import tpu as pltpu
from jax.experimental.pallas import tpu_sc as plsc

N = 10000
NPAD = 10240
D = 128
DH = 64
NG = 16
E = 320000
EPAD = 327680
NSUB = 16
EB = 128
NB = 160
RPT = NPAD // NSUB
BR = 1280
NBLK = NPAD // BR
F32 = jnp.float32


def _leaky(v):
    return jnp.where(v >= 0.0, v, v * 0.2)


def _store_logits(i, h, asc_ref, adc_ref, asr_ref, adr_ref,
                  as_col, ad_col, as_row, ad_row):
    arow = i * BR + lax.broadcasted_iota(jnp.int32, (BR, 1), 0)
    av = jnp.dot(h, asc_ref[...], preferred_element_type=F32)
    as_col[...] = jnp.where(arow < N, av, F32(-1e30))
    ad_col[...] = jnp.dot(h, adc_ref[...], preferred_element_type=F32)
    dn = (((1,), (1,)), ((), ()))
    avr = lax.dot_general(asr_ref[...], h, dn, preferred_element_type=F32)
    lrow = i * BR + lax.broadcasted_iota(jnp.int32, (1, BR), 1)
    sl = pl.ds(i * BR, BR)
    as_row[:, sl] = jnp.where(lrow < N, avr, F32(-1e30))
    ad_row[:, sl] = lax.dot_general(adr_ref[...], h, dn,
                                    preferred_element_type=F32)


def _embed_body(x_ref, w_ref, asc_ref, adc_ref, asr_ref, adr_ref,
                h_ref, as_col, ad_col, as_row, ad_row):
    h = jnp.dot(x_ref[...], w_ref[...], preferred_element_type=F32)
    h_ref[0] = h[:, 0:DH]
    h_ref[1] = h[:, DH:D]
    _store_logits(pl.program_id(0), h, asc_ref, adc_ref, asr_ref, adr_ref,
                  as_col, ad_col, as_row, ad_row)


_LOGIT_OUT_SPECS = [
    pl.BlockSpec((BR, 1), lambda i: (i, 0)),
    pl.BlockSpec((BR, 1), lambda i: (i, 0)),
    pl.BlockSpec((1, NPAD), lambda i: (0, 0)),
    pl.BlockSpec((1, NPAD), lambda i: (0, 0)),
]
_LOGIT_OUT_SHAPE = [
    jax.ShapeDtypeStruct((NPAD, 1), F32),
    jax.ShapeDtypeStruct((NPAD, 1), F32),
    jax.ShapeDtypeStruct((1, NPAD), F32),
    jax.ShapeDtypeStruct((1, NPAD), F32),
]
_VEC_IN_SPECS = [
    pl.BlockSpec((D, 1), lambda i: (0, 0)),
    pl.BlockSpec((D, 1), lambda i: (0, 0)),
    pl.BlockSpec((1, D), lambda i: (0, 0)),
    pl.BlockSpec((1, D), lambda i: (0, 0)),
]


def _tc_embed(xp, W, a_sc, a_dc, a_sr, a_dr):
    return pl.pallas_call(
        _embed_body,
        grid=(NBLK,),
        in_specs=[
            pl.BlockSpec((BR, D), lambda i: (i, 0)),
            pl.BlockSpec((D, D), lambda i: (0, 0)),
            *_VEC_IN_SPECS,
        ],
        out_specs=[
            pl.BlockSpec((2, BR, DH), lambda i: (0, i, 0)),
            *_LOGIT_OUT_SPECS,
        ],
        out_shape=[
            jax.ShapeDtypeStruct((2, NPAD, DH), F32),
            *_LOGIT_OUT_SHAPE,
        ],
    )(xp, W, a_sc, a_dc, a_sr, a_dr)


def _tables_body(as_ref, ad_ref, es_ref, m_ref):
    a = as_ref[...]
    d = ad_ref[...]
    m = jnp.max(a[0:N, :])
    c = _leaky(m + d)
    es_ref[...] = jnp.exp(_leaky(a + d) - c)
    m_ref[...] = jnp.full((1, 128), m, F32)


def _tc_tables(as1, ad1):
    return pl.pallas_call(
        _tables_body,
        out_shape=[
            jax.ShapeDtypeStruct((NPAD, 1), F32),
            jax.ShapeDtypeStruct((1, 128), F32),
        ],
    )(as1, ad1)


def _combine(o_ref, d_ref, h_ref, es_ref, b_ref):
    num = jnp.concatenate([o_ref[0], o_ref[1]], axis=1)
    h1 = jnp.concatenate([h_ref[0], h_ref[1]], axis=1)
    es = es_ref[...]
    num = num + es * h1
    den = d_ref[...] + es + 1e-16
    v = num / den + b_ref[...]
    return jnp.where(v > 0.0, v, jnp.exp(v) - 1.0)


def _comb_body(o_ref, d_ref, h_ref, es_ref, b_ref, w_ref,
               asc_ref, adc_ref, asr_ref, adr_ref,
               h2_ref, as_col, ad_col, as_row, ad_row):
    h1f = _combine(o_ref, d_ref, h_ref, es_ref, b_ref)
    h2 = jnp.dot(h1f, w_ref[...], preferred_element_type=F32)
    h2_ref[0] = h2[:, 0:DH]
    h2_ref[1] = h2[:, DH:D]
    _store_logits(pl.program_id(0), h2, asc_ref, adc_ref, asr_ref, adr_ref,
                  as_col, ad_col, as_row, ad_row)


def _tc_comb(o1, d1, h1p, es1, b, W, a_sc, a_dc, a_sr, a_dr):
    return pl.pallas_call(
        _comb_body,
        grid=(NBLK,),
        in_specs=[
            pl.BlockSpec((2, BR, DH), lambda i: (0, i, 0)),
            pl.BlockSpec((BR, 1), lambda i: (i, 0)),
            pl.BlockSpec((2, BR, DH), lambda i: (0, i, 0)),
            pl.BlockSpec((BR, 1), lambda i: (i, 0)),
            pl.BlockSpec((1, D), lambda i: (0, 0)),
            pl.BlockSpec((D, D), lambda i: (0, 0)),
            *_VEC_IN_SPECS,
        ],
        out_specs=[
            pl.BlockSpec((2, BR, DH), lambda i: (0, i, 0)),
            *_LOGIT_OUT_SPECS,
        ],
        out_shape=[
            jax.ShapeDtypeStruct((2, NPAD, DH), F32),
            *_LOGIT_OUT_SHAPE,
        ],
    )(o1, d1, h1p, es1, b, W, a_sc, a_dc, a_sr, a_dr)


def _pool_body(o_ref, d_ref, h_ref, es_ref, b_ref, bt_ref, sum_ref, cnt_ref):
    h2f = _combine(o_ref, d_ref, h_ref, es_ref, b_ref)
    bt = bt_ref[0]
    g = lax.broadcasted_iota(jnp.int32, (NG, BR), 0)
    oh = (g == bt).astype(F32)
    psum = jnp.dot(oh, h2f, preferred_element_type=F32)
    pcnt = jnp.broadcast_to(jnp.sum(oh, axis=1, keepdims=True), (NG, D))
    i = pl.program_id(0)

    @pl.when(i == 0)
    def _():
        sum_ref[...] = psum
        cnt_ref[...] = pcnt

    @pl.when(i != 0)
    def _():
        sum_ref[...] += psum
        cnt_ref[...] += pcnt


def _tc_pool(o2, d2, h2p, es2, b, batchp):
    return pl.pallas_call(
        _pool_body,
        grid=(NBLK,),
        in_specs=[
            pl.BlockSpec((2, BR, DH), lambda i: (0, i, 0)),
            pl.BlockSpec((BR, 1), lambda i: (i, 0)),
            pl.BlockSpec((2, BR, DH), lambda i: (0, i, 0)),
            pl.BlockSpec((BR, 1), lambda i: (i, 0)),
            pl.BlockSpec((1, D), lambda i: (0, 0)),
            pl.BlockSpec((1, 1, BR), lambda i: (i, 0, 0)),
        ],
        out_specs=[
            pl.BlockSpec((NG, D), lambda i: (0, 0)),
            pl.BlockSpec((NG, D), lambda i: (0, 0)),
        ],
        out_shape=[
            jax.ShapeDtypeStruct((NG, D), F32),
            jax.ShapeDtypeStruct((NG, D), F32),
        ],
    )(o2, d2, h2p, es2, b, batchp)


def _head_body(s_ref, c_ref, wc_ref, bc_ref, o_ref):
    cnt = c_ref[...][:, 0:1]
    pooled = s_ref[...] / jnp.maximum(cnt, 1.0)
    lg = jnp.dot(pooled, wc_ref[...], preferred_element_type=F32) + bc_ref[...]
    o_ref[...] = 1.0 / (1.0 + jnp.exp(-lg))


def _tc_head(sums, cnts, Wc, bc):
    return pl.pallas_call(
        _head_body,
        out_shape=jax.ShapeDtypeStruct((NG, 1), F32),
    )(sums, cnts, Wc, bc)


def _sc_body(h_hbm, src_hbm, dst_hbm, as_hbm, ad_hbm, m_hbm,
             out_hbm, den_hbm,
             src_v, dst_v, as_v, ad_v, m_v, r_a, r_b, s_buf,
             w_a, w_b, den_out,
             out_acc, den_acc, gs_a, gs_b, ss):
    cid = lax.axis_index("c")
    sid = lax.axis_index("s")

    pltpu.sync_copy(src_hbm.at[sid], src_v)
    pltpu.sync_copy(dst_hbm.at[sid], dst_v)
    pltpu.sync_copy(as_hbm.at[0], as_v)
    pltpu.sync_copy(ad_hbm.at[0], ad_v)
    pltpu.sync_copy(m_hbm.at[0, pl.ds(0, 16)], m_v)

    zero = jnp.zeros((16,), F32)

    def _zero_row(r, carry):
        for cc in range(DH // 16):
            r_a[r, pl.ds(cc * 16, 16)] = zero
        return carry

    lax.fori_loop(0, EB, _zero_row, 0)
    for cc in range(EB // 16):
        w_a[pl.ds(cc * 16, 16)] = zero

    base = sid * RPT
    for k in range(4):
        pltpu.sync_copy(r_a, out_acc.at[pl.ds(base + k * 128, 128)])
        pltpu.sync_copy(w_a, den_acc.at[pl.ds(base + k * 128, 128)])
    pltpu.sync_copy(r_a.at[pl.ds(0, RPT - 512)],
                    out_acc.at[pl.ds(base + 512, RPT - 512)])
    pltpu.sync_copy(w_a.at[pl.ds(0, RPT - 512)],
                    den_acc.at[pl.ds(base + 512, RPT - 512)])
    plsc.subcore_barrier()

    def compute_w(j, wref):
        mv = m_v[pl.ds(0, 16)]

        def wgroup(k, carry):
            sl = pl.ds(k * 16, 16)
            sv = src_v[j, sl]
            dv = dst_v[j, sl]
            a1 = plsc.load_gather(as_v, [sv])
            a2 = plsc.load_gather(ad_v, [dv])
            al = _leaky(a1 + a2)
            cg = _leaky(mv + a2)
            wref[sl] = jnp.exp(al - cg)
            return carry

        lax.fori_loop(0, EB // 16, wgroup, 0)

    def scale(rows, sbuf, wref):
        def sgroup(g, carry):
            wv = wref[pl.ds(g * 16, 16)]
            e0 = g * 16
            for l in range(16):
                we = jnp.take_along_axis(
                    wv, jnp.full((16,), l, jnp.int32), axis=0)
                for cc in range(DH // 16):
                    sl = pl.ds(cc * 16, 16)
                    sbuf[e0 + l, sl] = rows[e0 + l, sl] * we
            return carry

        lax.fori_loop(0, EB // 16, sgroup, 0)

    def gstart(j, rows, sem):
        pltpu.async_copy(h_hbm.at[cid].at[src_v.at[j]], rows, sem)

    def gwait(j, rows, sem):
        pltpu.make_async_copy(h_hbm.at[cid].at[src_v.at[j]], rows, sem).wait()

    def scat_start(j, wref):
        pltpu.async_copy(s_buf, out_acc.at[dst_v.at[j]], ss, add=True)

        @pl.when(cid == 0)
        def _():
            pltpu.async_copy(wref, den_acc.at[dst_v.at[j]], ss, add=True)

    def scat_wait(j, wref):
        pltpu.make_async_copy(s_buf, out_acc.at[dst_v.at[j]], ss).wait()

        @pl.when(cid == 0)
        def _():
            pltpu.make_async_copy(wref, den_acc.at[dst_v.at[j]], ss).wait()

    gstart(0, r_a, gs_a)
    gstart(1, r_b, gs_b)

    def body(jj, carry):
        j0 = jj * 2
        j1 = j0 + 1
        compute_w(j0, w_a)
        gwait(j0, r_a, gs_a)

        @pl.when(jj > 0)
        def _():
            scat_wait(j0 - 1, w_b)

        scale(r_a, s_buf, w_a)

        @pl.when(jj < NB // 2 - 1)
        def _():
            gstart(j0 + 2, r_a, gs_a)

        scat_start(j0, w_a)

        compute_w(j1, w_b)
        gwait(j1, r_b, gs_b)
        scat_wait(j0, w_a)
        scale(r_b, s_buf, w_b)

        @pl.when(jj < NB // 2 - 1)
        def _():
            gstart(j1 + 2, r_b, gs_b)

        scat_start(j1, w_b)

        return carry

    lax.fori_loop(0, NB // 2, body, 0)
    scat_wait(NB - 1, w_b)
    plsc.subcore_barrier()

    pltpu.sync_copy(out_acc.at[pl.ds(base, RPT)],
                    out_hbm.at[cid, pl.ds(base, RPT)])

    @pl.when(cid == 0)
    def _():
        pltpu.sync_copy(den_acc.at[pl.ds(base, RPT)], den_out)
        pltpu.sync_copy(den_out, den_hbm.at[pl.ds(base, RPT)])


@functools.partial(
    pl.kernel,
    out_type=[
        jax.ShapeDtypeStruct((2, NPAD, DH), F32),
        jax.ShapeDtypeStruct((NPAD,), F32),
    ],
    mesh=plsc.VectorSubcoreMesh(core_axis_name="c", subcore_axis_name="s"),
    compiler_params=pltpu.CompilerParams(
        needs_layout_passes=False, use_tc_tiling_on_sc=False),
    scratch_types=[
        pltpu.VMEM((NB, EB), jnp.int32),
        pltpu.VMEM((NB, EB), jnp.int32),
        pltpu.VMEM((NPAD,), F32),
        pltpu.VMEM((NPAD,), F32),
        pltpu.VMEM((16,), F32),
        pltpu.VMEM((EB, DH), F32),
        pltpu.VMEM((EB, DH), F32),
        pltpu.VMEM((EB, DH), F32),
        pltpu.VMEM((EB,), F32),
        pltpu.VMEM((EB,), F32),
        pltpu.VMEM((RPT,), F32),
        pltpu.VMEM_SHARED((NPAD, DH), F32),
        pltpu.VMEM_SHARED((NPAD,), F32),
        pltpu.SemaphoreType.DMA,
        pltpu.SemaphoreType.DMA,
        pltpu.SemaphoreType.DMA,
    ],
)
def _sc_edge(h_hbm, src_hbm, dst_hbm, as_hbm, ad_hbm, m_hbm,
             out_hbm, den_hbm, *rest):
    _sc_body(h_hbm, src_hbm, dst_hbm, as_hbm, ad_hbm, m_hbm,
             out_hbm, den_hbm, *rest)


def kernel(x, edge_index, batch, W1, a_src1, a_dst1, b1,
           W2, a_src2, a_dst2, b2, Wc, bc):
    src = edge_index[0]
    dst = edge_index[1]
    pad = (N + (jnp.arange(EPAD - E, dtype=jnp.int32) % (NPAD - N))).astype(jnp.int32)
    srcp = jnp.concatenate([src, pad]).reshape(NSUB, NB, EB)
    dstp = jnp.concatenate([dst, pad]).reshape(NSUB, NB, EB)
    xp = jnp.concatenate([x, jnp.zeros((NPAD - N, D), F32)])
    batchp = jnp.concatenate(
        [batch, jnp.full((NPAD - N,), NG, jnp.int32)]).reshape(NBLK, 1, BR)

    h1p, as1c, ad1c, as1r, ad1r = _tc_embed(
        xp, W1, a_src1.reshape(D, 1), a_dst1.reshape(D, 1), a_src1, a_dst1)
    es1, m1 = _tc_tables(as1c, ad1c)
    o1, d1 = _sc_edge(h1p, srcp, dstp, as1r, ad1r, m1)
    h2p, as2c, ad2c, as2r, ad2r = _tc_comb(
        o1, d1.reshape(NPAD, 1), h1p, es1, b1.reshape(1, D), W2,
        a_src2.reshape(D, 1), a_dst2.reshape(D, 1), a_src2, a_dst2)
    es2, m2 = _tc_tables(as2c, ad2c)
    o2, d2 = _sc_edge(h2p, srcp, dstp, as2r, ad2r, m2)
    sums, cnts = _tc_pool(o2, d2.reshape(NPAD, 1), h2p, es2,
                          b2.reshape(1, D), batchp)
    sig = _tc_head(sums, cnts, Wc, bc.reshape(1, 1))
    return sig.reshape(NG)
